# baseline (speedup 1.0000x reference)
import functools

import jax
import jax.numpy as jnp
from jax import lax
from jax.experimental import pallas as pl
from jax.experimental.pallas import tpu as pltpu
from jax.experimental.pallas import tpu_sc as plsc

B = 16384
EMBED_K = 64
FEAT = 128

NC = 2
NS = 16
NW = NC * NS
B_PER_W = B // NW
CHUNK = 128
N_CHUNKS = B_PER_W // CHUNK


def _sc_gather_body(users_hbm, items_hbm, tu_hbm, f_hbm, theta_out, effe_out,
                    uidx_v, iidx_v, urows_v, irows_v, usem, isem):
    wid = lax.axis_index("s") * NC + lax.axis_index("c")
    base = wid * B_PER_W
    for c in range(N_CHUNKS):
        off = base + c * CHUNK
        pltpu.sync_copy(users_hbm.at[pl.ds(off, CHUNK)], uidx_v)
        pltpu.sync_copy(items_hbm.at[pl.ds(off, CHUNK)], iidx_v)
        ucp = pltpu.async_copy(tu_hbm.at[uidx_v], urows_v, usem)
        icp = pltpu.async_copy(f_hbm.at[iidx_v], irows_v, isem)
        ucp.wait()
        icp.wait()
        pltpu.sync_copy(urows_v, theta_out.at[pl.ds(off, CHUNK)])
        pltpu.sync_copy(irows_v, effe_out.at[pl.ds(off, CHUNK)])


@jax.jit
def _sc_gather(users, items, tu, f):
    mesh = plsc.VectorSubcoreMesh(core_axis_name="c", subcore_axis_name="s")
    return pl.kernel(
        _sc_gather_body,
        out_type=(
            jax.ShapeDtypeStruct((B, EMBED_K), jnp.float32),
            jax.ShapeDtypeStruct((B, FEAT), jnp.float32),
        ),
        mesh=mesh,
        compiler_params=pltpu.CompilerParams(use_tc_tiling_on_sc=False),
        scratch_types=[
            pltpu.VMEM((CHUNK,), jnp.int32),
            pltpu.VMEM((CHUNK,), jnp.int32),
            pltpu.VMEM((CHUNK, EMBED_K), jnp.float32),
            pltpu.VMEM((CHUNK, FEAT), jnp.float32),
            pltpu.SemaphoreType.DMA,
            pltpu.SemaphoreType.DMA,
        ],
    )(users, items, tu, f)


TC_BLK = 2048


def _tc_body(gu_ref, gi_ref, th_ref, ef_ref, w_ref, b_ref, xui_ref, proj_ref):
    e = ef_ref[...]
    mm = lax.dot_general(e, w_ref[...], (((1,), (1,)), ((), ())),
                         preferred_element_type=jnp.float32)
    p = mm + b_ref[...]
    n = jnp.sqrt(jnp.sum(p * p, axis=1, keepdims=True))
    p = p / jnp.maximum(n, 1e-12)
    proj_ref[...] = p
    xui = (jnp.sum(gu_ref[...] * gi_ref[...], axis=1, keepdims=True)
           + jnp.sum(th_ref[...] * p, axis=1, keepdims=True))
    xui_ref[...] = xui


@jax.jit
def _tc_compute(gu, gi, theta_u, effe_i, w, b2d):
    grid = (B // TC_BLK,)
    xui2d, proj = pl.pallas_call(
        _tc_body,
        grid=grid,
        in_specs=[
            pl.BlockSpec((TC_BLK, EMBED_K), lambda i: (i, 0)),
            pl.BlockSpec((TC_BLK, EMBED_K), lambda i: (i, 0)),
            pl.BlockSpec((TC_BLK, EMBED_K), lambda i: (i, 0)),
            pl.BlockSpec((TC_BLK, FEAT), lambda i: (i, 0)),
            pl.BlockSpec((EMBED_K, FEAT), lambda i: (0, 0)),
            pl.BlockSpec((1, EMBED_K), lambda i: (0, 0)),
        ],
        out_specs=[
            pl.BlockSpec((TC_BLK, 1), lambda i: (i, 0)),
            pl.BlockSpec((TC_BLK, EMBED_K), lambda i: (i, 0)),
        ],
        out_shape=[
            jax.ShapeDtypeStruct((B, 1), jnp.float32),
            jax.ShapeDtypeStruct((B, EMBED_K), jnp.float32),
        ],
    )(gu, gi, theta_u, effe_i, w, b2d)
    return xui2d, proj


def kernel(gu, gi, users, items, Tu, F, W, b):
    users32 = users.astype(jnp.int32)
    items32 = items.astype(jnp.int32)
    theta_u, effe_i = _sc_gather(users32, items32, Tu, F)
    xui2d, proj_i = _tc_compute(gu, gi, theta_u, effe_i, W, b.reshape(1, EMBED_K))
    xui = xui2d.reshape(B)
    return (xui, gu, gi, theta_u, proj_i)
